# baseline (speedup 1.0000x reference)
import functools
import math

import jax
import jax.numpy as jnp
from jax import lax
from jax.experimental import pallas as pl
from jax.experimental.pallas import tpu as pltpu
from jax.experimental.pallas import tpu_sc as plsc

_EMB = 32
_SCALE = math.sqrt(_EMB)

_B = 16384 * 50
_NW = 32
_CHUNK = 128
_NCH = _B // (_NW * _CHUNK)


def _sc_gather(idx2d, table):
    mesh = plsc.VectorSubcoreMesh(core_axis_name="c", subcore_axis_name="s")

    @functools.partial(
        pl.kernel,
        mesh=mesh,
        out_type=jax.ShapeDtypeStruct((_B, _EMB), jnp.float32),
        scratch_types=[
            pltpu.VMEM((_NCH, _CHUNK), jnp.int32),
            pltpu.VMEM((_CHUNK, _EMB), jnp.float32),
            pltpu.SemaphoreType.DMA,
        ],
        compiler_params=pltpu.CompilerParams(use_tc_tiling_on_sc=False),
    )
    def k(idx_hbm, table_hbm, out_hbm, idx_v, rows_v, sem):
        wid = lax.axis_index("s") * 2 + lax.axis_index("c")
        row0 = wid * _NCH
        pltpu.sync_copy(idx_hbm.at[pl.ds(row0, _NCH)], idx_v)

        def chunk_body(j, carry):
            pltpu.async_copy(table_hbm.at[idx_v.at[j]], rows_v, sem).wait()

            def scale_body(r, c):
                rows_v[r, pl.ds(0, 16)] = rows_v[r, pl.ds(0, 16)] * _SCALE
                rows_v[r, pl.ds(16, 16)] = rows_v[r, pl.ds(16, 16)] * _SCALE
                return c

            lax.fori_loop(0, _CHUNK, scale_body, 0, unroll=4)
            pltpu.sync_copy(
                rows_v, out_hbm.at[pl.ds((row0 + j) * _CHUNK, _CHUNK)]
            )
            return carry

        lax.fori_loop(0, _NCH, chunk_body, 0)

    return k(idx2d, table)


def kernel(tokens, table):
    idx2d = tokens.reshape(_B // _CHUNK, _CHUNK).astype(jnp.int32)
    out = _sc_gather(idx2d, table)
    return out.reshape(tokens.shape[0], tokens.shape[1], _EMB)


# speedup vs baseline: 1.5722x; 1.5722x over previous
import functools
import math

import jax
import jax.numpy as jnp
from jax import lax
from jax.experimental import pallas as pl
from jax.experimental.pallas import tpu as pltpu
from jax.experimental.pallas import tpu_sc as plsc

_EMB = 32
_SCALE = math.sqrt(_EMB)

_S = 16384
_T = 50
_NW = 32
_TILES = _S // 128
_TPW = _TILES // _NW
_HT = 2
_NZ = _T * _TPW // _HT


def _sc_embed(idx3d, tpad):
    mesh = plsc.VectorSubcoreMesh(core_axis_name="c", subcore_axis_name="s")

    @functools.partial(
        pl.kernel,
        mesh=mesh,
        out_type=jax.ShapeDtypeStruct((_T, _EMB // 8, _TILES, 8, 128),
                                      jnp.float32),
        scratch_types=[
            pltpu.VMEM((_T, _TPW, 128), jnp.int32),
            pltpu.VMEM((128 * _HT, 128), jnp.float32),
            pltpu.VMEM((128 * _HT, 128), jnp.float32),
            pltpu.VMEM((_EMB // 8, _HT, 8, 128), jnp.float32),
            pltpu.VMEM((_EMB // 8, _HT, 8, 128), jnp.float32),
            pltpu.SemaphoreType.DMA,
            pltpu.SemaphoreType.DMA,
            pltpu.SemaphoreType.DMA,
            pltpu.SemaphoreType.DMA,
        ],
        compiler_params=pltpu.CompilerParams(use_tc_tiling_on_sc=False,
                                             needs_layout_passes=False),
    )
    def k(idx_hbm, table_hbm, out_hbm, idx_v, rows_a, rows_b, outs_a,
          outs_b, sem_a, sem_b, osem_a, osem_b):
        wid = lax.axis_index("s") * 2 + lax.axis_index("c")
        tile0 = wid * _TPW
        pltpu.sync_copy(idx_hbm.at[:, pl.ds(tile0, _TPW), :], idx_v)
        iota = lax.iota(jnp.int32, 16)
        evecs = [jnp.full((16,), e, jnp.int32) for e in range(_EMB)]

        def fire(z, rows, sem):
            t = z >> 1
            h = z & 1
            for c in range(_HT):
                pltpu.async_copy(table_hbm.at[idx_v.at[t, _HT * h + c]],
                                 rows.at[pl.ds(c * 128, 128)], sem)

        def drain(rows, sem):
            pltpu.make_async_copy(table_hbm.at[pl.ds(0, 128 * _HT)],
                                  rows, sem).wait()

        def out_slice(z):
            t = z >> 1
            h = z & 1
            return out_hbm.at[t, :, pl.ds(tile0 + _HT * h, _HT)]

        def compute(z, rows, outs, osem):
            @plsc.parallel_loop(0, 128 * _HT // 16, unroll=2)
            def comp(kk):
                s_vec = kk * 16 + iota
                c2 = kk >> 3
                lo = (kk & 7) * 16
                for g in range(_EMB // 8):
                    vals = []
                    for e in range(8 * g, 8 * g + 8):
                        vals.append(
                            plsc.load_gather(rows, [s_vec, evecs[e]]))
                    for r in range(8):
                        outs[g, c2, r, pl.ds(lo, 16)] = vals[r]

            pltpu.async_copy(outs, out_slice(z), osem)

        def odrain(z, outs, osem):
            pltpu.make_async_copy(outs, out_slice(z), osem).wait()

        fire(0, rows_a, sem_a)

        def z_body(zz, carry):
            z0 = 2 * zz
            drain(rows_a, sem_a)
            fire(z0 + 1, rows_b, sem_b)

            @pl.when(zz > 0)
            def _():
                odrain(z0 - 2, outs_a, osem_a)

            compute(z0, rows_a, outs_a, osem_a)
            drain(rows_b, sem_b)

            @pl.when(zz < _NZ // 2 - 1)
            def _():
                fire(z0 + 2, rows_a, sem_a)

            @pl.when(zz > 0)
            def _():
                odrain(z0 - 1, outs_b, osem_b)

            compute(z0 + 1, rows_b, outs_b, osem_b)
            return carry

        lax.fori_loop(0, _NZ // 2, z_body, 0)
        odrain(_NZ - 2, outs_a, osem_a)
        odrain(_NZ - 1, outs_b, osem_b)

    return k(idx3d, tpad)


def kernel(tokens, table):
    idx3d = jnp.swapaxes(tokens, 0, 1).reshape(_T, _TILES, 128)
    tpad = jnp.pad(table * jnp.float32(_SCALE), ((0, 0), (0, 128 - _EMB)))
    out5 = _sc_embed(idx3d.astype(jnp.int32), tpad)
    return out5.transpose(2, 4, 0, 1, 3).reshape(_S, _T, _EMB)
